# baseline (speedup 1.0000x reference)
import functools

import jax
import jax.numpy as jnp
from jax import lax
from jax.experimental import pallas as pl
from jax.experimental.pallas import tpu as pltpu
from jax.experimental.pallas import tpu_sc as plsc

_D = 64
_E = 1024
_BETA = 0.25
_T = 1024


def _dist_body(xt_ref, emb_ref, rn_ref, cn_ref, idx_ref, loss_ref, tab_ref):
    i = pl.program_id(0)
    xt = xt_ref[...]
    emb = emb_ref[...]

    @pl.when(i == 0)
    def _():
        embt = emb.T
        tab_ref[...] = jnp.concatenate([embt, embt], axis=1)
    sim = jnp.dot(xt, emb)
    rn = rn_ref[...]
    cn = cn_ref[...]
    d = (rn + cn) - 2.0 * sim
    dmin = jnp.min(d, axis=1, keepdims=True)
    iota = lax.broadcasted_iota(jnp.int32, d.shape, 1).astype(jnp.float32)
    idxf = jnp.min(jnp.where(d == dmin, iota, float(d.shape[1])), axis=1)
    idx_ref[...] = idxf[:, None].astype(jnp.int32)
    part = jnp.sum(dmin, axis=0, keepdims=True)

    @pl.when(i == 0)
    def _():
        loss_ref[...] = jnp.zeros_like(part)

    loss_ref[...] += part


_DP = 128
_CH = 128


def _make_sc_gather(m, n_cores, n_subcores):
    nw = n_cores * n_subcores
    b_per_w = m // nw
    nch = b_per_w // _CH
    mesh = plsc.VectorSubcoreMesh(core_axis_name="c", subcore_axis_name="s")

    @functools.partial(
        pl.kernel, mesh=mesh,
        out_type=jax.ShapeDtypeStruct((m * _D,), jnp.float32),
        scratch_types=[
            pltpu.VMEM((nch, _CH), jnp.int32),
            pltpu.VMEM((b_per_w, _DP), jnp.float32),
            pltpu.VMEM((b_per_w * _D,), jnp.float32),
            pltpu.SemaphoreType.DMA,
        ],
    )
    def _gather(table_hbm, idx_hbm, out_hbm, idx_v, rows_v, out_v, sem):
        wid = lax.axis_index("s") * n_cores + lax.axis_index("c")
        pltpu.sync_copy(idx_hbm.at[pl.ds(wid * nch, nch)], idx_v)
        copies = [
            pltpu.async_copy(table_hbm.at[idx_v.at[j]],
                             rows_v.at[pl.ds(j * _CH, _CH)], sem)
            for j in range(nch)
        ]
        for c in copies:
            c.wait()

        def _row(r, _):
            for k in range(_D // 16):
                out_v[pl.ds(r * _D + k * 16, 16)] = rows_v[r, pl.ds(k * 16, 16)]
            return _

        lax.fori_loop(0, b_per_w, _row, 0)
        pltpu.sync_copy(out_v, out_hbm.at[pl.ds(wid * b_per_w * _D,
                                                b_per_w * _D)])

    return _gather


def kernel(x, embeddings):
    input_shape = x.shape
    flat = x.reshape(-1, _D)
    m = flat.shape[0]
    grid = m // _T
    rownorm = jnp.sum(flat ** 2, axis=1, keepdims=True)
    colnorm = jnp.sum(embeddings ** 2, axis=0, keepdims=True)
    idx, loss_sum, table = pl.pallas_call(
        _dist_body,
        grid=(grid,),
        in_specs=[
            pl.BlockSpec((_T, _D), lambda i: (i, 0)),
            pl.BlockSpec((_D, _E), lambda i: (0, 0)),
            pl.BlockSpec((_T, 1), lambda i: (i, 0)),
            pl.BlockSpec((1, _E), lambda i: (0, 0)),
        ],
        out_specs=[
            pl.BlockSpec((_T, 1), lambda i: (i, 0)),
            pl.BlockSpec((1, 1), lambda i: (0, 0)),
            pl.BlockSpec((_E, _DP), lambda i: (0, 0)),
        ],
        out_shape=[
            jax.ShapeDtypeStruct((m, 1), jnp.int32),
            jax.ShapeDtypeStruct((1, 1), jnp.float32),
            jax.ShapeDtypeStruct((_E, _DP), jnp.float32),
        ],
    )(flat, embeddings, rownorm, colnorm)

    info = plsc.get_sparse_core_info()
    q = _make_sc_gather(m, info.num_cores, info.num_subcores)(
        table, idx.reshape(m // _CH, _CH))
    quantized = q.reshape(input_shape)
    loss = loss_sum[0, 0] * ((1.0 + _BETA) / (m * _D))
    return quantized, loss


# speedup vs baseline: 1.2046x; 1.2046x over previous
import functools

import jax
import jax.numpy as jnp
from jax import lax
from jax.experimental import pallas as pl
from jax.experimental.pallas import tpu as pltpu
from jax.experimental.pallas import tpu_sc as plsc

_D = 64
_E = 1024
_BETA = 0.25
_T = 2048


def _dist_body(xt_ref, emb_ref, rn_ref, cn_ref, idx_ref, loss_ref, tab_ref):
    i = pl.program_id(0)
    xt = xt_ref[...]
    emb = emb_ref[...]

    @pl.when(i == 0)
    def _():
        embt = emb.T
        tab_ref[...] = jnp.concatenate([embt, embt], axis=1)
    sim = jnp.dot(xt, emb)
    rn = rn_ref[...]
    cn = cn_ref[...]
    d = (rn + cn) - 2.0 * sim
    dmin = jnp.min(d, axis=1, keepdims=True)
    iota = lax.broadcasted_iota(jnp.int32, d.shape, 1).astype(jnp.float32)
    idxf = jnp.min(jnp.where(d == dmin, iota, float(d.shape[1])), axis=1)
    idx_ref[...] = idxf[:, None].astype(jnp.int32)
    part = jnp.sum(dmin, axis=0, keepdims=True)

    @pl.when(i == 0)
    def _():
        loss_ref[...] = jnp.zeros_like(part)

    loss_ref[...] += part


_DP = 128
_CH = 128


def _make_sc_gather(m, n_cores, n_subcores):
    nw = n_cores * n_subcores
    b_per_w = m // nw
    nch = b_per_w // _CH
    mesh = plsc.VectorSubcoreMesh(core_axis_name="c", subcore_axis_name="s")

    @functools.partial(
        pl.kernel, mesh=mesh,
        out_type=jax.ShapeDtypeStruct((m, _DP), jnp.float32),
        scratch_types=[
            pltpu.VMEM((nch, _CH), jnp.int32),
            pltpu.VMEM((b_per_w, _DP), jnp.float32),
            pltpu.SemaphoreType.DMA,
        ],
    )
    def _gather(table_hbm, idx_hbm, out_hbm, idx_v, rows_v, sem):
        wid = lax.axis_index("s") * n_cores + lax.axis_index("c")
        base = wid * b_per_w
        pltpu.sync_copy(idx_hbm.at[pl.ds(wid * nch, nch)], idx_v)
        copies = [
            pltpu.async_copy(table_hbm.at[idx_v.at[j]],
                             rows_v.at[pl.ds(j * _CH, _CH)], sem)
            for j in range(nch)
        ]
        for c in copies:
            c.wait()
        pltpu.sync_copy(rows_v, out_hbm.at[pl.ds(base, b_per_w)])

    return _gather


def kernel(x, embeddings):
    input_shape = x.shape
    flat = x.reshape(-1, _D)
    m = flat.shape[0]
    grid = m // _T
    rownorm = jnp.sum(flat ** 2, axis=1, keepdims=True)
    colnorm = jnp.sum(embeddings ** 2, axis=0, keepdims=True)
    idx, loss_sum, table = pl.pallas_call(
        _dist_body,
        grid=(grid,),
        in_specs=[
            pl.BlockSpec((_T, _D), lambda i: (i, 0)),
            pl.BlockSpec((_D, _E), lambda i: (0, 0)),
            pl.BlockSpec((_T, 1), lambda i: (i, 0)),
            pl.BlockSpec((1, _E), lambda i: (0, 0)),
        ],
        out_specs=[
            pl.BlockSpec((_T, 1), lambda i: (i, 0)),
            pl.BlockSpec((1, 1), lambda i: (0, 0)),
            pl.BlockSpec((_E, _DP), lambda i: (0, 0)),
        ],
        out_shape=[
            jax.ShapeDtypeStruct((m, 1), jnp.int32),
            jax.ShapeDtypeStruct((1, 1), jnp.float32),
            jax.ShapeDtypeStruct((_E, _DP), jnp.float32),
        ],
    )(flat, embeddings, rownorm, colnorm)

    info = plsc.get_sparse_core_info()
    qp = _make_sc_gather(m, info.num_cores, info.num_subcores)(
        table, idx.reshape(m // _CH, _CH))
    quantized = qp[:, :_D].reshape(input_shape)
    loss = loss_sum[0, 0] * ((1.0 + _BETA) / (m * _D))
    return quantized, loss


# speedup vs baseline: 1.2055x; 1.0007x over previous
import functools

import jax
import jax.numpy as jnp
from jax import lax
from jax.experimental import pallas as pl
from jax.experimental.pallas import tpu as pltpu
from jax.experimental.pallas import tpu_sc as plsc

_D = 64
_E = 1024
_BETA = 0.25
_T = 4096


def _dist_body(xt_ref, emb_ref, rn_ref, cn_ref, idx_ref, loss_ref, tab_ref):
    i = pl.program_id(0)
    xt = xt_ref[...]
    emb = emb_ref[...]

    @pl.when(i == 0)
    def _():
        embt = emb.T
        tab_ref[...] = jnp.concatenate([embt, embt], axis=1)
    sim = jnp.dot(xt, emb)
    rn = rn_ref[...]
    cn = cn_ref[...]
    d = (rn + cn) - 2.0 * sim
    dmin = jnp.min(d, axis=1, keepdims=True)
    iota = lax.broadcasted_iota(jnp.int32, d.shape, 1).astype(jnp.float32)
    idxf = jnp.min(jnp.where(d == dmin, iota, float(d.shape[1])), axis=1)
    idx_ref[...] = idxf[:, None].astype(jnp.int32)
    part = jnp.sum(dmin, axis=0, keepdims=True)

    @pl.when(i == 0)
    def _():
        loss_ref[...] = jnp.zeros_like(part)

    loss_ref[...] += part


_DP = 128
_CH = 128


def _make_sc_gather(m, n_cores, n_subcores):
    nw = n_cores * n_subcores
    b_per_w = m // nw
    nch = b_per_w // _CH
    mesh = plsc.VectorSubcoreMesh(core_axis_name="c", subcore_axis_name="s")

    @functools.partial(
        pl.kernel, mesh=mesh,
        out_type=jax.ShapeDtypeStruct((m, _DP), jnp.float32),
        scratch_types=[
            pltpu.VMEM((nch, _CH), jnp.int32),
            pltpu.VMEM((b_per_w, _DP), jnp.float32),
            pltpu.SemaphoreType.DMA,
        ],
    )
    def _gather(table_hbm, idx_hbm, out_hbm, idx_v, rows_v, sem):
        wid = lax.axis_index("s") * n_cores + lax.axis_index("c")
        base = wid * b_per_w
        pltpu.sync_copy(idx_hbm.at[pl.ds(wid * nch, nch)], idx_v)
        copies = [
            pltpu.async_copy(table_hbm.at[idx_v.at[j]],
                             rows_v.at[pl.ds(j * _CH, _CH)], sem)
            for j in range(nch)
        ]
        for c in copies:
            c.wait()
        pltpu.sync_copy(rows_v, out_hbm.at[pl.ds(base, b_per_w)])

    return _gather


def kernel(x, embeddings):
    input_shape = x.shape
    flat = x.reshape(-1, _D)
    m = flat.shape[0]
    grid = m // _T
    rownorm = jnp.sum(flat ** 2, axis=1, keepdims=True)
    colnorm = jnp.sum(embeddings ** 2, axis=0, keepdims=True)
    idx, loss_sum, table = pl.pallas_call(
        _dist_body,
        grid=(grid,),
        in_specs=[
            pl.BlockSpec((_T, _D), lambda i: (i, 0)),
            pl.BlockSpec((_D, _E), lambda i: (0, 0)),
            pl.BlockSpec((_T, 1), lambda i: (i, 0)),
            pl.BlockSpec((1, _E), lambda i: (0, 0)),
        ],
        out_specs=[
            pl.BlockSpec((_T, 1), lambda i: (i, 0)),
            pl.BlockSpec((1, 1), lambda i: (0, 0)),
            pl.BlockSpec((_E, _DP), lambda i: (0, 0)),
        ],
        out_shape=[
            jax.ShapeDtypeStruct((m, 1), jnp.int32),
            jax.ShapeDtypeStruct((1, 1), jnp.float32),
            jax.ShapeDtypeStruct((_E, _DP), jnp.float32),
        ],
    )(flat, embeddings, rownorm, colnorm)

    info = plsc.get_sparse_core_info()
    qp = _make_sc_gather(m, info.num_cores, info.num_subcores)(
        table, idx.reshape(m // _CH, _CH))
    quantized = qp[:, :_D].reshape(input_shape)
    loss = loss_sum[0, 0] * ((1.0 + _BETA) / (m * _D))
    return quantized, loss
